# baseline (speedup 1.0000x reference)
import jax
import jax.numpy as jnp
from jax.experimental import pallas as pl


def _broadcast_kernel(logps_ref, out_ref):
    out_ref[...] = jnp.broadcast_to(logps_ref[...][:, None, :], out_ref.shape)


def kernel(hist, logps):
    S, B = hist.shape
    V = logps.shape[0]
    logps2d = logps.reshape(1, V)

    out = pl.pallas_call(
        _broadcast_kernel,
        grid=(S + 1,),
        in_specs=[pl.BlockSpec((1, V), lambda i: (0, 0))],
        out_specs=pl.BlockSpec((1, B, V), lambda i: (i, 0, 0)),
        out_shape=jax.ShapeDtypeStruct((S + 1, B, V), jnp.float32),
    )(logps2d)
    return out
